# baseline (speedup 1.0000x reference)
import jax
import jax.numpy as jnp
from jax import lax
from jax.experimental import pallas as pl
from jax.experimental.pallas import tpu as pltpu
from jax.experimental.pallas import tpu_sc as plsc

_NTOK = 16384
_D = 64
_NC = 2
_NS = 16
_NW = _NC * _NS
_PER_W = _NTOK // _NW
_CHUNK = 128
_NCHUNK = _PER_W // _CHUNK
_LANES = 16
_DC = _D // _LANES


def _sc_body(ids_hbm, emb_hbm, out_hbm, idx_v, rows_v, acc_v, part_sh, sem):
    c = lax.axis_index("c")
    s = lax.axis_index("s")
    wid = s * _NC + c

    pltpu.sync_copy(ids_hbm.at[wid], idx_v)

    copies = []
    for k in range(_NCHUNK):
        copies.append(
            pltpu.async_copy(
                emb_hbm.at[idx_v.at[k]],
                rows_v.at[pl.ds(k * _CHUNK, _CHUNK)],
                sem,
            )
        )
    for cp in copies:
        cp.wait()

    def acc_step(j, carry):
        return tuple(
            carry[ci] + rows_v[j, pl.ds(ci * _LANES, _LANES)]
            for ci in range(_DC)
        )

    zeros = tuple(jnp.zeros((_LANES,), jnp.float32) for _ in range(_DC))
    accs = lax.fori_loop(0, _PER_W, acc_step, zeros)
    for ci in range(_DC):
        acc_v[pl.ds(ci * _LANES, _LANES)] = accs[ci]

    pltpu.sync_copy(acc_v, part_sh.at[s])
    plsc.subcore_barrier()

    @pl.when(s == 0)
    def _tail():
        part_v = rows_v.at[pl.ds(0, _NS)]
        pltpu.sync_copy(part_sh, part_v)

        def red_step(j, carry):
            return tuple(
                carry[ci] + part_v[j, pl.ds(ci * _LANES, _LANES)]
                for ci in range(_DC)
            )

        totals = lax.fori_loop(0, _NS, red_step, zeros)
        for ci in range(_DC):
            acc_v[pl.ds(ci * _LANES, _LANES)] = totals[ci]
        pltpu.sync_copy(acc_v, out_hbm.at[c])


def _tc_body(part_ref, wt_ref, o_ref):
    pooled = jnp.sum(part_ref[...], axis=0, keepdims=True) * (1.0 / _NTOK)
    o_ref[...] = jnp.dot(pooled, wt_ref[...],
                         preferred_element_type=jnp.float32)


@jax.jit
def _run(ids, embedding, wt):
    mesh = plsc.VectorSubcoreMesh(core_axis_name="c", subcore_axis_name="s")
    sums = pl.kernel(
        _sc_body,
        out_type=jax.ShapeDtypeStruct((_NC, _D), jnp.float32),
        mesh=mesh,
        scratch_types=[
            pltpu.VMEM((_NCHUNK, _CHUNK), jnp.int32),
            pltpu.VMEM((_PER_W, _D), jnp.float32),
            pltpu.VMEM((_D,), jnp.float32),
            pltpu.VMEM_SHARED((_NS, _D), jnp.float32),
            pltpu.SemaphoreType.DMA,
        ],
        compiler_params=pltpu.CompilerParams(use_tc_tiling_on_sc=False),
        name="input_adapter_sc",
    )(ids, embedding)
    out = pl.pallas_call(
        _tc_body,
        out_shape=jax.ShapeDtypeStruct((1, _D), jnp.float32),
        name="input_adapter_tc_tail",
    )(sums, wt)
    return out


def kernel(token_ids, embedding, W):
    ids = token_ids.astype(jnp.int32).reshape(_NW, _NCHUNK, _CHUNK)
    return _run(ids, embedding, W.T)


# speedup vs baseline: 1.6431x; 1.6431x over previous
import jax
import jax.numpy as jnp
from jax import lax
from jax.experimental import pallas as pl
from jax.experimental.pallas import tpu as pltpu
from jax.experimental.pallas import tpu_sc as plsc

_NTOK = 16384
_D = 64
_NC = 2
_NS = 16
_NW = _NC * _NS
_PER_W = _NTOK // _NW
_LANES = 16
_NGRP = _PER_W // _LANES
_DC = _D // _LANES


def _sc_body(ids_hbm, emb_hbm, out_hbm, idx_v, rows_v, acc_v, sem):
    c = lax.axis_index("c")
    s = lax.axis_index("s")
    wid = s * _NC + c

    pltpu.sync_copy(ids_hbm.at[pl.ds(wid * _PER_W, _PER_W)], idx_v)

    zeros = tuple(jnp.zeros((_LANES,), jnp.float32) for _ in range(_DC))

    def group(g, carry):
        vec = idx_v[pl.ds(g * _LANES, _LANES)]
        base = g * _LANES
        copies = [
            pltpu.async_copy(
                emb_hbm.at[vec[l]], rows_v.at[base + l], sem
            )
            for l in range(_LANES)
        ]
        for cp in copies:
            cp.wait()

        def acc_step(j, acc):
            return tuple(
                acc[ci] + rows_v[j, pl.ds(ci * _LANES, _LANES)]
                for ci in range(_DC)
            )

        return lax.fori_loop(base, base + _LANES, acc_step, carry)

    accs = lax.fori_loop(0, _NGRP, group, zeros)
    for ci in range(_DC):
        acc_v[pl.ds(ci * _LANES, _LANES)] = accs[ci]

    pltpu.sync_copy(acc_v, out_hbm.at[wid])


def _tc_body(part_ref, wt_ref, o_ref):
    pooled = jnp.sum(part_ref[...], axis=0, keepdims=True) * (1.0 / _NTOK)
    o_ref[...] = jnp.dot(pooled, wt_ref[...],
                         preferred_element_type=jnp.float32)


@jax.jit
def _run(ids, embedding, wt):
    mesh = plsc.VectorSubcoreMesh(core_axis_name="c", subcore_axis_name="s")
    sums = pl.kernel(
        _sc_body,
        out_type=jax.ShapeDtypeStruct((_NW, _D), jnp.float32),
        mesh=mesh,
        scratch_types=[
            pltpu.VMEM((_PER_W,), jnp.int32),
            pltpu.VMEM((_PER_W, _D), jnp.float32),
            pltpu.VMEM((_D,), jnp.float32),
            pltpu.SemaphoreType.DMA,
        ],
        name="input_adapter_sc",
    )(ids, embedding)
    out = pl.pallas_call(
        _tc_body,
        out_shape=jax.ShapeDtypeStruct((1, _D), jnp.float32),
        name="input_adapter_tc_tail",
    )(sums, wt)
    return out


def kernel(token_ids, embedding, W):
    return _run(token_ids.astype(jnp.int32), embedding, W.T)


# speedup vs baseline: 4.4400x; 2.7021x over previous
import jax
import jax.numpy as jnp
from jax import lax
from jax.experimental import pallas as pl
from jax.experimental.pallas import tpu as pltpu
from jax.experimental.pallas import tpu_sc as plsc

_NTOK = 16384
_D = 64
_VOCAB = 1000000
_NC = 2
_NS = 16
_NW = _NC * _NS
_PER_W = _NTOK // _NW
_CHUNK = 128
_NCHUNK = _PER_W // _CHUNK
_LANES = 16
_HPAD = 1000064
_ZCH = 62528
_ZLAST = _HPAD - 15 * _ZCH
_C = 16128
_NMAIN = 62 * _C
_TAIL = _VOCAB - _NMAIN


def _hist_body(ids_hbm, zeros_hbm, out_hbm, idx_v, vals_v, zbuf_v, hist_sh):
    c = lax.axis_index("c")
    s = lax.axis_index("s")
    wid = s * _NC + c

    pltpu.sync_copy(ids_hbm.at[wid], idx_v)

    for ci in range(_CHUNK // _LANES):
        vals_v[pl.ds(ci * _LANES, _LANES)] = jnp.full((_LANES,), 1.0,
                                                      jnp.float32)

    pltpu.sync_copy(zeros_hbm, zbuf_v)

    @pl.when(s < _NS - 1)
    def _zmain():
        pltpu.sync_copy(zbuf_v, hist_sh.at[pl.ds(s * _ZCH, _ZCH)])

    @pl.when(s == _NS - 1)
    def _zlast():
        pltpu.sync_copy(zbuf_v.at[pl.ds(0, _ZLAST)],
                        hist_sh.at[pl.ds(15 * _ZCH, _ZLAST)])

    plsc.subcore_barrier()

    for k in range(_NCHUNK):
        pltpu.sync_copy(vals_v, hist_sh.at[idx_v.at[k]], add=True)
    plsc.subcore_barrier()

    @pl.when(s < _NS - 1)
    def _dmain():
        pltpu.sync_copy(hist_sh.at[pl.ds(s * _ZCH, _ZCH)], zbuf_v)
        pltpu.sync_copy(zbuf_v,
                        out_hbm.at[pl.ds(c * _HPAD + s * _ZCH, _ZCH)])

    @pl.when(s == _NS - 1)
    def _dlast():
        zpart = zbuf_v.at[pl.ds(0, _ZLAST)]
        pltpu.sync_copy(hist_sh.at[pl.ds(15 * _ZCH, _ZLAST)], zpart)
        pltpu.sync_copy(
            zpart, out_hbm.at[pl.ds(c * _HPAD + 15 * _ZCH, _ZLAST)]
        )


def _scan_body(tbl_ref, cnt_ref, o_ref):
    i = pl.program_id(0)

    @pl.when(i == 0)
    def _init():
        o_ref[...] = jnp.zeros_like(o_ref)

    csum = cnt_ref[0, :] + cnt_ref[1, :]
    o_ref[...] += jnp.dot(
        tbl_ref[...], csum, preferred_element_type=jnp.float32
    )[None, :]


def _finish_body(main_ref, ctail_ref, ttail_ref, wt_ref, o_ref):
    ct = ctail_ref[0, :] + ctail_ref[1, :]
    tail = jnp.dot(ttail_ref[...], ct, preferred_element_type=jnp.float32)
    pooled = (main_ref[0, :] + tail) * (1.0 / _NTOK)
    o_ref[...] = jnp.dot(pooled[None, :], wt_ref[...],
                         preferred_element_type=jnp.float32)


@jax.jit
def _run(ids, emb_t, wt):
    mesh = plsc.VectorSubcoreMesh(core_axis_name="c", subcore_axis_name="s")
    counts1d = pl.kernel(
        _hist_body,
        out_type=jax.ShapeDtypeStruct((_NC * _HPAD,), jnp.float32),
        mesh=mesh,
        scratch_types=[
            pltpu.VMEM((_NCHUNK, _CHUNK), jnp.int32),
            pltpu.VMEM((_CHUNK,), jnp.float32),
            pltpu.VMEM((_ZCH,), jnp.float32),
            pltpu.VMEM_SHARED((_HPAD,), jnp.float32),
        ],
        name="token_histogram_sc",
    )(ids, jnp.zeros((_ZCH,), jnp.float32))
    counts = counts1d.reshape(_NC, _HPAD)

    main = pl.pallas_call(
        _scan_body,
        grid=(_NMAIN // _C,),
        in_specs=[
            pl.BlockSpec((_D, _C), lambda i: (0, i)),
            pl.BlockSpec((_NC, _C), lambda i: (0, i)),
        ],
        out_specs=pl.BlockSpec((1, _D), lambda i: (0, 0)),
        out_shape=jax.ShapeDtypeStruct((1, _D), jnp.float32),
        name="table_scan_matvec_tc",
    )(emb_t, counts)

    ctail = lax.slice(counts, (0, _NMAIN), (_NC, _VOCAB))
    ttail = lax.slice(emb_t, (0, _NMAIN), (_D, _VOCAB))
    out = pl.pallas_call(
        _finish_body,
        out_shape=jax.ShapeDtypeStruct((1, _D), jnp.float32),
        name="finish_tc",
    )(main, ctail, ttail, wt)
    return out


def kernel(token_ids, embedding, W):
    ids = token_ids.astype(jnp.int32).reshape(_NW, _NCHUNK, _CHUNK)
    return _run(ids, embedding.T, W.T)


# speedup vs baseline: 4.8252x; 1.0868x over previous
import jax
import jax.numpy as jnp
from jax import lax
from jax.experimental import pallas as pl
from jax.experimental.pallas import tpu as pltpu
from jax.experimental.pallas import tpu_sc as plsc

_NTOK = 16384
_D = 64
_VOCAB = 1000000
_NC = 2
_NS = 16
_NW = _NC * _NS
_PER_W = _NTOK // _NW
_CHUNK = 128
_NCHUNK = _PER_W // _CHUNK
_LANES = 16
_HPAD = 1000064
_ZCH = 62592
_ZLAST = _HPAD - 15 * _ZCH
_C = 16128
_NMAIN = 62 * _C
_TAIL = _VOCAB - _NMAIN


def _hist_body(ids_hbm, zeros_hbm, out_hbm, idx_v, vals_v, zbuf_v, hist_sh):
    c = lax.axis_index("c")
    s = lax.axis_index("s")
    wid = s * _NC + c

    pltpu.sync_copy(ids_hbm.at[wid], idx_v)

    for ci in range(_CHUNK // _LANES):
        vals_v[pl.ds(ci * _LANES, _LANES)] = jnp.full((_LANES,), 1.0,
                                                      jnp.float32)

    pltpu.sync_copy(zeros_hbm, zbuf_v)

    @pl.when(s < _NS - 1)
    def _zmain():
        pltpu.sync_copy(zbuf_v, hist_sh.at[pl.ds(s * _ZCH, _ZCH)])

    @pl.when(s == _NS - 1)
    def _zlast():
        pltpu.sync_copy(zbuf_v.at[pl.ds(0, _ZLAST)],
                        hist_sh.at[pl.ds(15 * _ZCH, _ZLAST)])

    plsc.subcore_barrier()

    for k in range(_NCHUNK):
        pltpu.sync_copy(vals_v, hist_sh.at[idx_v.at[k]], add=True)
    plsc.subcore_barrier()

    @pl.when(s < _NS - 1)
    def _dmain():
        pltpu.sync_copy(hist_sh.at[pl.ds(s * _ZCH, _ZCH)], zbuf_v)
        pltpu.sync_copy(zbuf_v, out_hbm.at[c, pl.ds(s * _ZCH, _ZCH)])

    @pl.when(s == _NS - 1)
    def _dlast():
        zpart = zbuf_v.at[pl.ds(0, _ZLAST)]
        pltpu.sync_copy(hist_sh.at[pl.ds(15 * _ZCH, _ZLAST)], zpart)
        pltpu.sync_copy(zpart, out_hbm.at[c, pl.ds(15 * _ZCH, _ZLAST)])


def _scan_body(tbl_ref, cnt_ref, o_ref):
    i = pl.program_id(0)

    @pl.when(i == 0)
    def _init():
        o_ref[...] = jnp.zeros_like(o_ref)

    csum = cnt_ref[0, :] + cnt_ref[1, :]
    o_ref[...] += jnp.dot(
        tbl_ref[...], csum, preferred_element_type=jnp.float32
    )[None, :]


def _finish_body(main_ref, ctail_ref, ttail_ref, wt_ref, o_ref):
    ct = ctail_ref[0, :] + ctail_ref[1, :]
    tail = jnp.dot(ttail_ref[...], ct, preferred_element_type=jnp.float32)
    pooled = (main_ref[0, :] + tail) * (1.0 / _NTOK)
    o_ref[...] = jnp.dot(pooled[None, :], wt_ref[...],
                         preferred_element_type=jnp.float32)


@jax.jit
def _run(ids, emb_t, wt):
    mesh = plsc.VectorSubcoreMesh(core_axis_name="c", subcore_axis_name="s")
    counts = pl.kernel(
        _hist_body,
        out_type=jax.ShapeDtypeStruct((_NC, _HPAD), jnp.float32),
        mesh=mesh,
        scratch_types=[
            pltpu.VMEM((_NCHUNK, _CHUNK), jnp.int32),
            pltpu.VMEM((_CHUNK,), jnp.float32),
            pltpu.VMEM((_ZCH,), jnp.float32),
            pltpu.VMEM_SHARED((_HPAD,), jnp.float32),
        ],
        name="token_histogram_sc",
    )(ids, jnp.zeros((_ZCH,), jnp.float32))

    main = pl.pallas_call(
        _scan_body,
        grid=(_NMAIN // _C,),
        in_specs=[
            pl.BlockSpec((_D, _C), lambda i: (0, i)),
            pl.BlockSpec((_NC, _C), lambda i: (0, i)),
        ],
        out_specs=pl.BlockSpec((1, _D), lambda i: (0, 0)),
        out_shape=jax.ShapeDtypeStruct((1, _D), jnp.float32),
        name="table_scan_matvec_tc",
    )(emb_t, counts)

    ctail = lax.slice(counts, (0, _NMAIN), (_NC, _VOCAB))
    ttail = lax.slice(emb_t, (0, _NMAIN), (_D, _VOCAB))
    out = pl.pallas_call(
        _finish_body,
        out_shape=jax.ShapeDtypeStruct((1, _D), jnp.float32),
        name="finish_tc",
    )(main, ctail, ttail, wt)
    return out


def kernel(token_ids, embedding, W):
    ids = token_ids.astype(jnp.int32).reshape(_NW, _NCHUNK, _CHUNK)
    return _run(ids, embedding.T, W.T)
